# baseline (speedup 1.0000x reference)
import functools

import jax
import jax.numpy as jnp
from jax import lax
from jax.experimental import pallas as pl
from jax.experimental.pallas import tpu as pltpu
from jax.experimental.pallas import tpu_sc as plsc

VOCAB_SIZE = 1000
EMBED_D = 8
N_TOKENS = 1024 * 50

NUM_CORES = 2
NUM_SUBCORES = 16
NW = NUM_CORES * NUM_SUBCORES
PER_W = N_TOKENS // NW
CHUNK = 40
NCH = PER_W // CHUNK


def _table_body(embed_ref, wt_ref, b_ref, table_ref):
    table_ref[...] = (
        jnp.dot(embed_ref[...], wt_ref[...], preferred_element_type=jnp.float32)
        + b_ref[...]
    )


def _make_table(embed, wt, b2d):
    return pl.pallas_call(
        _table_body,
        out_shape=jax.ShapeDtypeStruct((VOCAB_SIZE, VOCAB_SIZE), jnp.float32),
    )(embed, wt, b2d)


_SC_MESH = plsc.VectorSubcoreMesh(
    core_axis_name="c",
    subcore_axis_name="s",
    num_cores=NUM_CORES,
    num_subcores=NUM_SUBCORES,
)


@functools.partial(
    pl.kernel,
    mesh=_SC_MESH,
    compiler_params=pltpu.CompilerParams(use_tc_tiling_on_sc=False),
    out_type=jax.ShapeDtypeStruct((N_TOKENS, VOCAB_SIZE), jnp.float32),
    scratch_types=[
        pltpu.VMEM((NCH, CHUNK), jnp.int32),
        pltpu.VMEM((2, CHUNK, VOCAB_SIZE), jnp.float32),
        pltpu.SemaphoreType.DMA,
        pltpu.SemaphoreType.DMA,
        pltpu.SemaphoreType.DMA,
        pltpu.SemaphoreType.DMA,
    ],
)
def _gather_rows(table_hbm, idx_hbm, out_hbm, idx_v, rows_v, g0, g1, s0, s1):
    wid = lax.axis_index("s") * NUM_CORES + lax.axis_index("c")
    base = wid * PER_W
    gsems = (g0, g1)
    ssems = (s0, s1)

    pltpu.sync_copy(idx_hbm.at[wid], idx_v)

    pltpu.async_copy(table_hbm.at[idx_v.at[0]], rows_v.at[0], gsems[0])
    pltpu.async_copy(table_hbm.at[idx_v.at[1]], rows_v.at[1], gsems[1])

    @pl.loop(0, NCH, step=2)
    def _pair(j):
        for slot in range(2):
            cj = j + slot
            pltpu.make_async_copy(
                table_hbm.at[idx_v.at[cj]], rows_v.at[slot], gsems[slot]
            ).wait()
            pltpu.async_copy(
                rows_v.at[slot],
                out_hbm.at[pl.ds(base + cj * CHUNK, CHUNK)],
                ssems[slot],
            ).wait()

            @pl.when(cj + 2 < NCH)
            def _refill():
                pltpu.async_copy(
                    table_hbm.at[idx_v.at[cj + 2]], rows_v.at[slot], gsems[slot]
                )


def kernel(input_ids, embed, W, b):
    B, S = input_ids.shape
    table = _make_table(embed, W.T, b.reshape(1, VOCAB_SIZE))
    ids = input_ids.reshape(NW, NCH, CHUNK).astype(jnp.int32)
    out = _gather_rows(table, ids)
    return out.reshape(B, S, VOCAB_SIZE)
